# baseline (speedup 1.0000x reference)
import functools

import jax
import jax.numpy as jnp
from jax import lax
from jax.experimental import pallas as pl
from jax.experimental.pallas import tpu as pltpu
from jax.experimental.pallas import tpu_sc as plsc

NC = 2
NS = 16
NW = NC * NS
L = 16

B = 16384
D = 64
NPW = B // NW
CHUNKS = D // L

_mesh = functools.partial(
    plsc.VectorSubcoreMesh, core_axis_name="c", subcore_axis_name="s"
)

_params = pltpu.CompilerParams(use_tc_tiling_on_sc=False)
_params_nolayout = pltpu.CompilerParams(
    use_tc_tiling_on_sc=False, needs_layout_passes=False
)


@functools.partial(
    pl.kernel,
    out_type=(
        jax.ShapeDtypeStruct((NW * L,), jnp.float32),
        jax.ShapeDtypeStruct((B,), jnp.float32),
    ),
    mesh=_mesh(),
    scratch_types=[
        pltpu.VMEM((NPW,), jnp.int32),
        pltpu.VMEM((NPW,), jnp.int32),
        pltpu.VMEM((NPW, D), jnp.float32),
        pltpu.VMEM((NPW, D), jnp.float32),
        pltpu.VMEM((NPW,), jnp.float32),
        pltpu.VMEM((NPW,), jnp.float32),
        pltpu.VMEM((L,), jnp.float32),
        pltpu.SemaphoreType.DMA,
        pltpu.SemaphoreType.DMA,
        pltpu.SemaphoreType.DMA,
        pltpu.SemaphoreType.DMA,
    ],
    compiler_params=_params,
)
def _gather_dot(uidx_hbm, sidx_hbm, uemb_hbm, ubias_hbm, semb_hbm, sbias_hbm,
                part_out, bsum_out,
                uidx_v, sidx_v, urows_v, srows_v, ub_v, sb_v, acc_v,
                sem_u, sem_s, sem_ub, sem_sb):
    wid = lax.axis_index("s") * NC + lax.axis_index("c")
    base = wid * NPW

    pltpu.sync_copy(uidx_hbm.at[pl.ds(base, NPW)], uidx_v)
    pltpu.sync_copy(sidx_hbm.at[pl.ds(base, NPW)], sidx_v)

    cp_u = pltpu.async_copy(uemb_hbm.at[uidx_v], urows_v, sem_u)
    cp_s = pltpu.async_copy(semb_hbm.at[sidx_v], srows_v, sem_s)
    cp_ub = pltpu.async_copy(ubias_hbm.at[uidx_v], ub_v, sem_ub)
    cp_sb = pltpu.async_copy(sbias_hbm.at[sidx_v], sb_v, sem_sb)

    cp_ub.wait()
    cp_sb.wait()

    def bias_body(i, _):
        s = pl.ds(i * L, L)
        ub_v[s] = ub_v[s] + sb_v[s]
        return 0

    lax.fori_loop(0, NPW // L, bias_body, 0)
    pltpu.sync_copy(ub_v, bsum_out.at[pl.ds(base, NPW)])

    cp_u.wait()
    cp_s.wait()

    def dot_body(r, acc):
        for c in range(CHUNKS):
            s = pl.ds(c * L, L)
            acc = acc + urows_v[r, s] * srows_v[r, s]
        return acc

    acc = lax.fori_loop(0, NPW, dot_body, jnp.zeros((L,), jnp.float32))
    acc_v[...] = acc
    pltpu.sync_copy(acc_v, part_out.at[pl.ds(wid * L, L)])


@functools.partial(
    pl.kernel,
    out_type=jax.ShapeDtypeStruct((B,), jnp.float32),
    mesh=_mesh(),
    scratch_types=[
        pltpu.VMEM((NW * L,), jnp.float32),
        pltpu.VMEM((NPW,), jnp.float32),
    ],
    compiler_params=_params_nolayout,
)
def _finish(part_hbm, bsum_hbm, out_hbm, part_v, b_v):
    wid = lax.axis_index("s") * NC + lax.axis_index("c")
    base = wid * NPW

    pltpu.sync_copy(part_hbm, part_v)
    pltpu.sync_copy(bsum_hbm.at[pl.ds(base, NPW)], b_v)

    def red_body(i, acc):
        return acc + part_v[pl.ds(i * L, L)]

    acc = lax.fori_loop(0, NW, red_body, jnp.zeros((L,), jnp.float32))
    total = jnp.sum(acc)

    def sig_body(i, _):
        s = pl.ds(i * L, L)
        x = b_v[s] + total
        b_v[s] = 1.0 / (1.0 + jnp.exp(-x))
        return 0

    lax.fori_loop(0, NPW // L, sig_body, 0)
    pltpu.sync_copy(b_v, out_hbm.at[pl.ds(base, NPW)])


def kernel(inputs, user_emb, user_bias, song_emb, song_bias):
    u_idx = inputs[:, 0].astype(jnp.int32)
    s_idx = inputs[:, 1].astype(jnp.int32)
    part, bsum = _gather_dot(
        u_idx, s_idx,
        user_emb, user_bias.reshape(-1),
        song_emb, song_bias.reshape(-1),
    )
    out = _finish(part, bsum)
    return out.reshape(B, 1)


# speedup vs baseline: 1.1951x; 1.1951x over previous
import functools

import jax
import jax.numpy as jnp
from jax import lax
from jax.experimental import pallas as pl
from jax.experimental.pallas import tpu as pltpu
from jax.experimental.pallas import tpu_sc as plsc

NC = 2
NS = 16
NW = NC * NS
L = 16

B = 16384
D = 64
NPW = B // NW
HALF = NPW // 2

_mesh = functools.partial(
    plsc.VectorSubcoreMesh, core_axis_name="c", subcore_axis_name="s"
)

_params_finish = pltpu.CompilerParams(
    use_tc_tiling_on_sc=False, needs_layout_passes=False
)


W = 1024


def _make_repack(n_rows):
    grid = (n_rows + 2 * W - 1) // (2 * W)

    def body(in_ref, out_ref):
        x = in_ref[...]
        out_ref[...] = jnp.concatenate(
            [x[:, :W].T, x[:, W:].T], axis=1
        )

    return pl.pallas_call(
        body,
        grid=(grid,),
        in_specs=[pl.BlockSpec((D, 2 * W), lambda i: (0, i))],
        out_specs=pl.BlockSpec((W, 2 * D), lambda i: (i, 0)),
        out_shape=jax.ShapeDtypeStruct((grid * W, 2 * D), jnp.float32),
    )


_repack_user = _make_repack(100000)
_repack_song = _make_repack(1000000)


@functools.partial(
    pl.kernel,
    out_type=jax.ShapeDtypeStruct((NW * L,), jnp.float32),
    mesh=_mesh(),
    scratch_types=[
        pltpu.VMEM((NPW,), jnp.int32),
        pltpu.VMEM((NPW,), jnp.int32),
        pltpu.VMEM((NPW,), jnp.int32),
        pltpu.VMEM((NPW,), jnp.int32),
        pltpu.VMEM((HALF, 2 * D), jnp.float32),
        pltpu.VMEM((HALF, 2 * D), jnp.float32),
        pltpu.VMEM((L,), jnp.float32),
        pltpu.SemaphoreType.DMA,
        pltpu.SemaphoreType.DMA,
    ],
)
def _gather_dot(ugidx_hbm, sgidx_hbm, uoff_hbm, soff_hbm, u2_hbm, s2_hbm,
                part_out,
                ugidx_v, sgidx_v, uoff_v, soff_v, urows_v, srows_v, acc_v,
                sem_u, sem_s):
    wid = lax.axis_index("s") * NC + lax.axis_index("c")
    base = wid * NPW

    pltpu.sync_copy(ugidx_hbm.at[pl.ds(base, NPW)], ugidx_v)
    pltpu.sync_copy(sgidx_hbm.at[pl.ds(base, NPW)], sgidx_v)
    pltpu.sync_copy(uoff_hbm.at[pl.ds(base, NPW)], uoff_v)
    pltpu.sync_copy(soff_hbm.at[pl.ds(base, NPW)], soff_v)

    def half_dot(h, acc):
        rbase = h * HALF
        cp_u = pltpu.async_copy(
            u2_hbm.at[ugidx_v.at[pl.ds(rbase, HALF)]], urows_v, sem_u)
        cp_s = pltpu.async_copy(
            s2_hbm.at[sgidx_v.at[pl.ds(rbase, HALF)]], srows_v, sem_s)
        cp_u.wait()
        cp_s.wait()

        def dot_body(g, a):
            uo16 = uoff_v[pl.ds(rbase + g * L, L)]
            so16 = soff_v[pl.ds(rbase + g * L, L)]
            for j in range(L):
                r = g * L + j
                uo = uo16[j]
                so = so16[j]
                for c in range(D // L):
                    a = a + (urows_v[r, pl.ds(uo + c * L, L)]
                             * srows_v[r, pl.ds(so + c * L, L)])
            return a

        return lax.fori_loop(0, HALF // L, dot_body, acc)

    acc = lax.fori_loop(0, 2, half_dot, jnp.zeros((L,), jnp.float32))
    acc_v[...] = acc
    pltpu.sync_copy(acc_v, part_out.at[pl.ds(wid * L, L)])


@functools.partial(
    pl.kernel,
    out_type=jax.ShapeDtypeStruct((B,), jnp.float32),
    mesh=_mesh(),
    scratch_types=[
        pltpu.VMEM((NW * L,), jnp.float32),
        pltpu.VMEM((NPW,), jnp.int32),
        pltpu.VMEM((NPW,), jnp.int32),
        pltpu.VMEM((NPW,), jnp.float32),
        pltpu.VMEM((NPW,), jnp.float32),
        pltpu.SemaphoreType.DMA,
        pltpu.SemaphoreType.DMA,
    ],
    compiler_params=_params_finish,
)
def _finish(part_hbm, uidx_hbm, sidx_hbm, ubias_hbm, sbias_hbm, out_hbm,
            part_v, uidx_v, sidx_v, ub_v, sb_v, sem_ub, sem_sb):
    wid = lax.axis_index("s") * NC + lax.axis_index("c")
    base = wid * NPW

    pltpu.sync_copy(uidx_hbm.at[pl.ds(base, NPW)], uidx_v)
    pltpu.sync_copy(sidx_hbm.at[pl.ds(base, NPW)], sidx_v)
    cp_ub = pltpu.async_copy(ubias_hbm.at[uidx_v], ub_v, sem_ub)
    cp_sb = pltpu.async_copy(sbias_hbm.at[sidx_v], sb_v, sem_sb)

    pltpu.sync_copy(part_hbm, part_v)

    def red_body(i, acc):
        return acc + part_v[pl.ds(i * L, L)]

    acc = lax.fori_loop(0, NW, red_body, jnp.zeros((L,), jnp.float32))
    total = jnp.sum(acc)

    cp_ub.wait()
    cp_sb.wait()

    def sig_body(i, _):
        s = pl.ds(i * L, L)
        x = ub_v[s] + sb_v[s] + total
        ub_v[s] = 1.0 / (1.0 + jnp.exp(-x))
        return 0

    lax.fori_loop(0, NPW // L, sig_body, 0)
    pltpu.sync_copy(ub_v, out_hbm.at[pl.ds(base, NPW)])


def kernel(inputs, user_emb, user_bias, song_emb, song_bias):
    u_idx = inputs[:, 0].astype(jnp.int32)
    s_idx = inputs[:, 1].astype(jnp.int32)

    u2 = _repack_user(user_emb.T)
    s2 = _repack_song(song_emb.T)

    ugidx = ((u_idx >> 11) << 10) + (u_idx & (W - 1))
    uoff = ((u_idx >> 10) & 1) * D
    sgidx = ((s_idx >> 11) << 10) + (s_idx & (W - 1))
    soff = ((s_idx >> 10) & 1) * D

    part = _gather_dot(ugidx, sgidx, uoff, soff, u2, s2)
    out = _finish(part, u_idx, s_idx,
                  user_bias.reshape(-1), song_bias.reshape(-1))
    return out.reshape(B, 1)
